# baseline (speedup 1.0000x reference)
import functools

import jax
import jax.numpy as jnp
from jax import lax
from jax.experimental import pallas as pl
from jax.experimental.pallas import tpu as pltpu
from jax.experimental.pallas import tpu_sc as plsc

N = 100000
D = 5
P = 8
L = 20
NC = 2
NS = 16
NW = NC * NS
BLK = 128
KC = 56
NBUF = 8
GLA = 4
NB = 784
TOTB = NB * NW
EPAD = TOTB * BLK
PAD_ROWS = 1120
NPAD = N + PAD_ROWS
RPT = NPAD // NS
BLK_TC = 6320


def _edge_body(xs_hbm, src_hbm, dst_hbm, zero_hbm, out_hbm,
               acc_sm, xs_sm, sidx, didx, rows, gsems, ssems):
    c = lax.axis_index("c")
    s = lax.axis_index("s")
    w = c * NS + s
    acc_s = acc_sm
    pltpu.sync_copy(xs_hbm.at[pl.ds(s * RPT, RPT)], xs_sm.at[pl.ds(s * RPT, RPT)])
    pltpu.sync_copy(zero_hbm.at[pl.ds(s * RPT, RPT)], acc_s.at[pl.ds(s * RPT, RPT)])
    plsc.subcore_barrier()

    def _gather(k):
        return pltpu.async_copy(xs_hbm.at[sidx.at[k]], rows.at[k % NBUF],
                                gsems.at[k % NBUF])

    def _scatter(k):
        return pltpu.async_copy(rows.at[k % NBUF], acc_s.at[didx.at[k]],
                                ssems.at[k % NBUF], add=True)

    def chunk(ci, _):
        cb = pl.multiple_of(w * NB + ci * KC, 8)
        pltpu.sync_copy(src_hbm.at[pl.ds(cb, KC)], sidx)
        pltpu.sync_copy(dst_hbm.at[pl.ds(cb, KC)], didx)
        g_descs = [None] * KC
        s_descs = [None] * KC
        for k in range(KC):
            if k >= NBUF:
                s_descs[k - NBUF].wait()
            g_descs[k] = _gather(k)
            if k >= GLA:
                g_descs[k - GLA].wait()
                s_descs[k - GLA] = _scatter(k - GLA)
        for k in range(KC - GLA, KC):
            g_descs[k].wait()
            s_descs[k] = _scatter(k)
        for k in range(KC - NBUF, KC):
            s_descs[k].wait()
        return 0

    lax.fori_loop(0, NB // KC, chunk, 0)
    plsc.subcore_barrier()
    pltpu.sync_copy(acc_s.at[pl.ds(s * RPT, RPT)],
                    out_hbm.at[c, pl.ds(s * RPT, RPT)])


_edge_pass = functools.partial(
    pl.kernel,
    out_type=jax.ShapeDtypeStruct((NC, NPAD, P), jnp.float32),
    mesh=plsc.VectorSubcoreMesh(core_axis_name="c", subcore_axis_name="s"),
    scratch_types=[
        pltpu.VMEM_SHARED((NPAD, P), jnp.float32),
        pltpu.VMEM_SHARED((NPAD, P), jnp.float32),
        pltpu.VMEM((KC, BLK), jnp.int32),
        pltpu.VMEM((KC, BLK), jnp.int32),
        pltpu.VMEM((NBUF, BLK, P), jnp.float32),
        pltpu.SemaphoreType.DMA((NBUF,)),
        pltpu.SemaphoreType.DMA((NBUF,)),
    ],
    compiler_params=pltpu.CompilerParams(use_tc_tiling_on_sc=False),
)(_edge_body)


def _row_mask(blk_idx):
    rows = blk_idx * BLK_TC + lax.broadcasted_iota(jnp.int32, (BLK_TC, P), 0)
    return rows < N


def _tc0_body(accd_ref, h_ref, w0_ref, dis_ref, xs_ref):
    deg = accd_ref[0] + accd_ref[1] + 1.0
    dis = jnp.where(_row_mask(pl.program_id(0)), lax.rsqrt(deg), 0.0)
    dis_ref[...] = dis
    xs_ref[...] = (h_ref[...] @ w0_ref[...]) * dis


def _tcmid_body(acc_ref, xs_ref, dis_ref, b_ref, w_ref, out_ref):
    dis = dis_ref[...]
    h = jax.nn.relu(dis * (acc_ref[0] + acc_ref[1] + xs_ref[...]) + b_ref[...])
    out_ref[...] = (h @ w_ref[...]) * dis


def _tcfin_body(acc_ref, xs_ref, dis_ref, b_ref, fw_ref, fb_ref, out_ref):
    dis = dis_ref[...]
    h = jax.nn.relu(dis * (acc_ref[0] + acc_ref[1] + xs_ref[...]) + b_ref[...])
    h = jax.nn.relu(h @ fw_ref[0] + fb_ref[0, :][None, :])
    h = jax.nn.relu(h @ fw_ref[1] + fb_ref[1, :][None, :])
    out_ref[...] = h @ fw_ref[2] + fb_ref[2, :][None, :]


_GRID = (NPAD // BLK_TC,)
_ACC_SPEC = pl.BlockSpec((NC, BLK_TC, P), lambda i: (0, i, 0))
_ROW_SPEC = pl.BlockSpec((BLK_TC, P), lambda i: (i, 0))
_B_SPEC = pl.BlockSpec((1, P), lambda i: (0, 0))
_W_SPEC = pl.BlockSpec((P, P), lambda i: (0, 0))
_FW_SPEC = pl.BlockSpec((3, P, P), lambda i: (0, 0, 0))
_FB_SPEC = pl.BlockSpec((3, P), lambda i: (0, 0))
_ROW_TY = jax.ShapeDtypeStruct((NPAD, P), jnp.float32)

_tc0 = pl.pallas_call(
    _tc0_body, grid=_GRID,
    in_specs=[_ACC_SPEC, _ROW_SPEC, _W_SPEC],
    out_specs=[_ROW_SPEC, _ROW_SPEC],
    out_shape=[_ROW_TY, _ROW_TY],
)

_tcmid = pl.pallas_call(
    _tcmid_body, grid=_GRID,
    in_specs=[_ACC_SPEC, _ROW_SPEC, _ROW_SPEC, _B_SPEC, _W_SPEC],
    out_specs=_ROW_SPEC,
    out_shape=_ROW_TY,
)

_tcfin = pl.pallas_call(
    _tcfin_body, grid=_GRID,
    in_specs=[_ACC_SPEC, _ROW_SPEC, _ROW_SPEC, _B_SPEC, _FW_SPEC, _FB_SPEC],
    out_specs=_ROW_SPEC,
    out_shape=_ROW_TY,
)


def kernel(h, edge_index, edge_weight, conv_W, conv_b, fc_W, fc_b):
    f32 = jnp.float32
    src = edge_index[0]
    dst = edge_index[1]
    npad_e = EPAD - src.shape[0]
    pad_idx = N + (jnp.arange(npad_e, dtype=jnp.int32) % 1024)
    src_p = jnp.concatenate([src, pad_idx]).reshape(TOTB, BLK)
    dst_p = jnp.concatenate([dst, pad_idx]).reshape(TOTB, BLK)

    row_ids = jnp.arange(NPAD, dtype=jnp.int32)[:, None]
    ones_t = jnp.where(row_ids < N, 1.0, 0.0).astype(f32) * jnp.ones((1, P), f32)
    zeros_t = jnp.zeros((NPAD, P), f32)
    h_p = jnp.zeros((NPAD, P), f32).at[:N, :D].set(h)

    wp = jnp.zeros((L, P, P), f32).at[:, :D, :D].set(conv_W)
    bp = jnp.zeros((L, P), f32).at[:, :D].set(conv_b)
    fwp = jnp.zeros((3, P, P), f32).at[:, :D, :D].set(fc_W)
    fbp = jnp.zeros((3, P), f32).at[:, :D].set(fc_b)

    acc_deg = _edge_pass(ones_t, src_p, dst_p, zeros_t)
    dis, xs = _tc0(acc_deg, h_p, wp[0])
    for l in range(L - 1):
        acc = _edge_pass(xs, src_p, dst_p, zeros_t)
        xs = _tcmid(acc, xs, dis, bp[l][None, :], wp[l + 1])
    acc = _edge_pass(xs, src_p, dst_p, zeros_t)
    out = _tcfin(acc, xs, dis, bp[L - 1][None, :], fwp, fbp)
    return out[:N, :D]


# speedup vs baseline: 1.3356x; 1.3356x over previous
import functools

import jax
import jax.numpy as jnp
from jax import lax
from jax.experimental import pallas as pl
from jax.experimental.pallas import tpu as pltpu
from jax.experimental.pallas import tpu_sc as plsc

N = 100000
D = 5
P = 8
L = 20
NC = 2
NS = 16
NW = NC * NS
BLK = 128
KC = 56
NBUF = 8
GLA = 4
NB = 784
TOTB = NB * NW
EPAD = TOTB * BLK
PAD_ROWS = 1120
NPAD = N + PAD_ROWS
RPT = NPAD // NS
BLK_TC = 6320


def _edge_body(xs_hbm, src_hbm, dst_hbm, zero_hbm, out_hbm,
               acc_sm, xs_sm, sidx, didx, rows, gsems, ssems):
    c = lax.axis_index("c")
    s = lax.axis_index("s")
    w = c * NS + s
    acc_s = acc_sm
    pltpu.sync_copy(xs_hbm.at[pl.ds(s * RPT, RPT)], xs_sm.at[pl.ds(s * RPT, RPT)])
    pltpu.sync_copy(zero_hbm.at[pl.ds(s * RPT, RPT)], acc_s.at[pl.ds(s * RPT, RPT)])
    plsc.subcore_barrier()

    def _gather(k):
        return pltpu.async_copy(xs_sm.at[sidx.at[k]], rows.at[k % NBUF],
                                gsems.at[k % NBUF])

    def _scatter(k):
        return pltpu.async_copy(rows.at[k % NBUF], acc_s.at[didx.at[k]],
                                ssems.at[k % NBUF], add=True)

    def chunk(ci, _):
        cb = pl.multiple_of(w * NB + ci * KC, 8)
        pltpu.sync_copy(src_hbm.at[pl.ds(cb, KC)], sidx)
        pltpu.sync_copy(dst_hbm.at[pl.ds(cb, KC)], didx)
        g_descs = [None] * KC
        s_descs = [None] * KC
        for k in range(KC):
            if k >= NBUF:
                s_descs[k - NBUF].wait()
            g_descs[k] = _gather(k)
            if k >= GLA:
                g_descs[k - GLA].wait()
                s_descs[k - GLA] = _scatter(k - GLA)
        for k in range(KC - GLA, KC):
            g_descs[k].wait()
            s_descs[k] = _scatter(k)
        for k in range(KC - NBUF, KC):
            s_descs[k].wait()
        return 0

    lax.fori_loop(0, NB // KC, chunk, 0)
    plsc.subcore_barrier()
    pltpu.sync_copy(acc_s.at[pl.ds(s * RPT, RPT)],
                    out_hbm.at[c, pl.ds(s * RPT, RPT)])


def _deg_body(dst_hbm, zero_hbm, ones_hbm, out_hbm,
              acc_sm, didx, ones_v, ssems):
    c = lax.axis_index("c")
    s = lax.axis_index("s")
    w = c * NS + s
    acc_s = acc_sm
    pltpu.sync_copy(zero_hbm.at[pl.ds(s * RPT, RPT)], acc_s.at[pl.ds(s * RPT, RPT)])
    pltpu.sync_copy(ones_hbm, ones_v)
    plsc.subcore_barrier()

    def chunk(ci, _):
        cb = pl.multiple_of(w * NB + ci * KC, 8)
        pltpu.sync_copy(dst_hbm.at[pl.ds(cb, KC)], didx)
        s_descs = [None] * KC
        for k in range(KC):
            if k >= NBUF:
                s_descs[k - NBUF].wait()
            s_descs[k] = pltpu.async_copy(ones_v, acc_s.at[didx.at[k]],
                                          ssems.at[k % NBUF], add=True)
        for k in range(KC - NBUF, KC):
            s_descs[k].wait()
        return 0

    lax.fori_loop(0, NB // KC, chunk, 0)
    plsc.subcore_barrier()
    pltpu.sync_copy(acc_s.at[pl.ds(s * RPT, RPT)],
                    out_hbm.at[c, pl.ds(s * RPT, RPT)])


_deg_pass = functools.partial(
    pl.kernel,
    out_type=jax.ShapeDtypeStruct((NC, NPAD, P), jnp.float32),
    mesh=plsc.VectorSubcoreMesh(core_axis_name="c", subcore_axis_name="s"),
    scratch_types=[
        pltpu.VMEM_SHARED((NPAD, P), jnp.float32),
        pltpu.VMEM((KC, BLK), jnp.int32),
        pltpu.VMEM((BLK, P), jnp.float32),
        pltpu.SemaphoreType.DMA((NBUF,)),
    ],
    compiler_params=pltpu.CompilerParams(use_tc_tiling_on_sc=False),
)(_deg_body)


_edge_pass = functools.partial(
    pl.kernel,
    out_type=jax.ShapeDtypeStruct((NC, NPAD, P), jnp.float32),
    mesh=plsc.VectorSubcoreMesh(core_axis_name="c", subcore_axis_name="s"),
    scratch_types=[
        pltpu.VMEM_SHARED((NPAD, P), jnp.float32),
        pltpu.VMEM_SHARED((NPAD, P), jnp.float32),
        pltpu.VMEM((KC, BLK), jnp.int32),
        pltpu.VMEM((KC, BLK), jnp.int32),
        pltpu.VMEM((NBUF, BLK, P), jnp.float32),
        pltpu.SemaphoreType.DMA((NBUF,)),
        pltpu.SemaphoreType.DMA((NBUF,)),
    ],
    compiler_params=pltpu.CompilerParams(use_tc_tiling_on_sc=False),
)(_edge_body)


def _row_mask(blk_idx):
    rows = blk_idx * BLK_TC + lax.broadcasted_iota(jnp.int32, (BLK_TC, P), 0)
    return rows < N


def _tc0_body(accd_ref, h_ref, w0_ref, dis_ref, xs_ref):
    deg = accd_ref[0] + accd_ref[1] + 1.0
    dis = jnp.where(_row_mask(pl.program_id(0)), lax.rsqrt(deg), 0.0)
    dis_ref[...] = dis
    xs_ref[...] = (h_ref[...] @ w0_ref[...]) * dis


def _tcmid_body(acc_ref, xs_ref, dis_ref, b_ref, w_ref, out_ref):
    dis = dis_ref[...]
    h = jax.nn.relu(dis * (acc_ref[0] + acc_ref[1] + xs_ref[...]) + b_ref[...])
    out_ref[...] = (h @ w_ref[...]) * dis


def _tcfin_body(acc_ref, xs_ref, dis_ref, b_ref, fw_ref, fb_ref, out_ref):
    dis = dis_ref[...]
    h = jax.nn.relu(dis * (acc_ref[0] + acc_ref[1] + xs_ref[...]) + b_ref[...])
    h = jax.nn.relu(h @ fw_ref[0] + fb_ref[0, :][None, :])
    h = jax.nn.relu(h @ fw_ref[1] + fb_ref[1, :][None, :])
    out_ref[...] = h @ fw_ref[2] + fb_ref[2, :][None, :]


_GRID = (NPAD // BLK_TC,)
_ACC_SPEC = pl.BlockSpec((NC, BLK_TC, P), lambda i: (0, i, 0))
_ROW_SPEC = pl.BlockSpec((BLK_TC, P), lambda i: (i, 0))
_B_SPEC = pl.BlockSpec((1, P), lambda i: (0, 0))
_W_SPEC = pl.BlockSpec((P, P), lambda i: (0, 0))
_FW_SPEC = pl.BlockSpec((3, P, P), lambda i: (0, 0, 0))
_FB_SPEC = pl.BlockSpec((3, P), lambda i: (0, 0))
_ROW_TY = jax.ShapeDtypeStruct((NPAD, P), jnp.float32)

_tc0 = pl.pallas_call(
    _tc0_body, grid=_GRID,
    in_specs=[_ACC_SPEC, _ROW_SPEC, _W_SPEC],
    out_specs=[_ROW_SPEC, _ROW_SPEC],
    out_shape=[_ROW_TY, _ROW_TY],
)

_tcmid = pl.pallas_call(
    _tcmid_body, grid=_GRID,
    in_specs=[_ACC_SPEC, _ROW_SPEC, _ROW_SPEC, _B_SPEC, _W_SPEC],
    out_specs=_ROW_SPEC,
    out_shape=_ROW_TY,
)

_tcfin = pl.pallas_call(
    _tcfin_body, grid=_GRID,
    in_specs=[_ACC_SPEC, _ROW_SPEC, _ROW_SPEC, _B_SPEC, _FW_SPEC, _FB_SPEC],
    out_specs=_ROW_SPEC,
    out_shape=_ROW_TY,
)


def kernel(h, edge_index, edge_weight, conv_W, conv_b, fc_W, fc_b):
    f32 = jnp.float32
    src = edge_index[0]
    dst = edge_index[1]
    npad_e = EPAD - src.shape[0]
    pad_idx = N + (jnp.arange(npad_e, dtype=jnp.int32) % 1024)
    src_p = jnp.concatenate([src, pad_idx]).reshape(TOTB, BLK)
    dst_p = jnp.concatenate([dst, pad_idx]).reshape(TOTB, BLK)

    ones_blk = jnp.ones((BLK, P), f32)
    zeros_t = jnp.zeros((NPAD, P), f32)
    h_p = jnp.zeros((NPAD, P), f32).at[:N, :D].set(h)

    wp = jnp.zeros((L, P, P), f32).at[:, :D, :D].set(conv_W)
    bp = jnp.zeros((L, P), f32).at[:, :D].set(conv_b)
    fwp = jnp.zeros((3, P, P), f32).at[:, :D, :D].set(fc_W)
    fbp = jnp.zeros((3, P), f32).at[:, :D].set(fc_b)

    acc_deg = _deg_pass(dst_p, zeros_t, ones_blk)
    dis, xs = _tc0(acc_deg, h_p, wp[0])
    for l in range(L - 1):
        acc = _edge_pass(xs, src_p, dst_p, zeros_t)
        xs = _tcmid(acc, xs, dis, bp[l][None, :], wp[l + 1])
    acc = _edge_pass(xs, src_p, dst_p, zeros_t)
    out = _tcfin(acc, xs, dis, bp[L - 1][None, :], fwp, fbp)
    return out[:N, :D]
